# Initial kernel scaffold; baseline (speedup 1.0000x reference)
import jax
import jax.numpy as jnp
from jax.experimental import pallas as pl


def kernel(x, adj_t, edge_attr, emb_ea, W0, b0, W1, b1, W2, b2):
    raise NotImplementedError("write your pallas kernel here")


# speedup vs baseline: 8.6507x; 8.6507x over previous
import functools

import jax
import jax.numpy as jnp
from jax import lax
from jax.experimental import pallas as pl
from jax.experimental.pallas import tpu as pltpu
from jax.experimental.pallas import tpu_sc as plsc

N = 10000
E = 320000
D = 128
NC = 2
NS = 16
NW = NC * NS
EPW = E // NW
CH = 80
NCH = EPW // CH
NP = 10240
RP = NP // NS

_mesh = plsc.VectorSubcoreMesh(core_axis_name="c", subcore_axis_name="s")


def _wid():
    return lax.axis_index("s") * NC + lax.axis_index("c")


@functools.partial(
    pl.kernel,
    out_type=jax.ShapeDtypeStruct((NW * N,), jnp.float32),
    mesh=_mesh,
    compiler_params=pltpu.CompilerParams(needs_layout_passes=False),
    scratch_types=[pltpu.VMEM((EPW,), jnp.int32),
                   pltpu.VMEM((N,), jnp.float32)],
)
def _sc_degree(col_hbm, degp_hbm, col_v, hist_v):
    wid = _wid()
    pltpu.sync_copy(col_hbm.at[pl.ds(wid * EPW, EPW)], col_v)

    def zbody(i, c):
        hist_v[pl.ds(i * 16, 16)] = jnp.zeros((16,), jnp.float32)
        return c
    lax.fori_loop(0, N // 16, zbody, 0)

    ones = jnp.ones((16,), jnp.float32)

    def body(i, c):
        idx = col_v[pl.ds(i * 16, 16)]
        plsc.addupdate_scatter(hist_v, [idx], ones)
        return c
    lax.fori_loop(0, EPW // 16, body, 0)
    pltpu.sync_copy(hist_v, degp_hbm.at[pl.ds(wid * N, N)])


@functools.partial(
    pl.kernel,
    out_type=jax.ShapeDtypeStruct((E,), jnp.float32),
    mesh=_mesh,
    compiler_params=pltpu.CompilerParams(needs_layout_passes=False),
    scratch_types=[pltpu.VMEM((N,), jnp.float32),
                   pltpu.VMEM((EPW,), jnp.int32),
                   pltpu.VMEM((EPW,), jnp.float32)],
)
def _sc_drow(row_hbm, dinv_hbm, drow_hbm, dv, rowi, dro):
    wid = _wid()
    pltpu.sync_copy(dinv_hbm, dv)
    pltpu.sync_copy(row_hbm.at[pl.ds(wid * EPW, EPW)], rowi)

    def body(i, c):
        idx = rowi[pl.ds(i * 16, 16)]
        dro[pl.ds(i * 16, 16)] = plsc.load_gather(dv, [idx])
        return c
    lax.fori_loop(0, EPW // 16, body, 0)
    pltpu.sync_copy(dro, drow_hbm.at[pl.ds(wid * EPW, EPW)])


def _make_sc_scatter(gather_mode):
    @functools.partial(
        pl.kernel,
        out_type=jax.ShapeDtypeStruct((NC * NP, D), jnp.float32),
        mesh=_mesh,
        compiler_params=pltpu.CompilerParams(needs_layout_passes=False),
        scratch_types=[pltpu.VMEM((NCH, CH), jnp.int32),
                       pltpu.VMEM((NCH, CH), jnp.int32),
                       pltpu.VMEM((CH, D), jnp.float32),
                       pltpu.VMEM_SHARED((NP, D), jnp.float32),
                       pltpu.SemaphoreType.DMA],
    )
    def _sc_scatter(src_hbm, row2_hbm, col2_hbm, zeros_hbm, out_hbm,
                    rowi_v, coli_v, rows_v, acc_sh, sem):
        cid = lax.axis_index("c")
        sid = lax.axis_index("s")
        wid = sid * NC + cid
        if gather_mode:
            pltpu.sync_copy(row2_hbm.at[wid], rowi_v)
        pltpu.sync_copy(col2_hbm.at[wid], coli_v)
        pltpu.sync_copy(zeros_hbm, acc_sh.at[pl.ds(sid * RP, RP)])
        plsc.subcore_barrier()

        def body(i, c):
            if gather_mode:
                pltpu.async_copy(src_hbm.at[rowi_v.at[i]], rows_v, sem).wait()
            else:
                pltpu.sync_copy(src_hbm.at[pl.ds(wid * EPW + i * CH, CH)],
                                rows_v)
            pltpu.sync_copy(rows_v, acc_sh.at[coli_v.at[i]], add=True)
            return c
        lax.fori_loop(0, NCH, body, 0)
        plsc.subcore_barrier()
        pltpu.sync_copy(acc_sh.at[pl.ds(sid * RP, RP)],
                        out_hbm.at[pl.ds(cid * NP + sid * RP, RP)])

    return _sc_scatter


_sc_scatter_gather = _make_sc_scatter(True)
_sc_scatter_linear = _make_sc_scatter(False)


BN = 2000
BE = 2000


def _tc_dinv_body(degp_ref, dinv_ref):
    deg = jnp.sum(degp_ref[...], axis=0)
    dinv_ref[...] = jnp.where(deg > 0, lax.rsqrt(deg), 0.0)[None, :]


def _tc_dinv(degp):
    return pl.pallas_call(
        _tc_dinv_body,
        out_shape=jax.ShapeDtypeStruct((1, N), jnp.float32),
    )(degp)


def _tc_z_body(ea_ref, emb_ref, dr_ref, z_ref):
    z_ref[...] = jnp.dot(ea_ref[...], emb_ref[...],
                         preferred_element_type=jnp.float32) * dr_ref[...]


def _tc_z(edge_attr, emb_ea, drow):
    return pl.pallas_call(
        _tc_z_body,
        grid=(E // BE,),
        in_specs=[pl.BlockSpec((BE, 16), lambda i: (i, 0)),
                  pl.BlockSpec((16, D), lambda i: (0, 0)),
                  pl.BlockSpec((BE, 1), lambda i: (i, 0))],
        out_specs=pl.BlockSpec((BE, D), lambda i: (i, 0)),
        out_shape=jax.ShapeDtypeStruct((E, D), jnp.float32),
    )(edge_attr, emb_ea, drow.reshape(E, 1))


def _tc_lin_body(h_ref, wt_ref, b_ref, dinv_ref, xp_ref, y_ref):
    xp = jnp.dot(h_ref[...], wt_ref[...],
                 preferred_element_type=jnp.float32) + b_ref[...]
    xp_ref[...] = xp
    y_ref[...] = xp * dinv_ref[...]


def _tc_lin(h, Wt, b2, dinvc):
    return pl.pallas_call(
        _tc_lin_body,
        grid=(N // BN,),
        in_specs=[pl.BlockSpec((BN, D), lambda i: (i, 0)),
                  pl.BlockSpec((D, D), lambda i: (0, 0)),
                  pl.BlockSpec((1, D), lambda i: (0, 0)),
                  pl.BlockSpec((BN, 1), lambda i: (i, 0))],
        out_specs=[pl.BlockSpec((BN, D), lambda i: (i, 0)),
                   pl.BlockSpec((BN, D), lambda i: (i, 0))],
        out_shape=[jax.ShapeDtypeStruct((N, D), jnp.float32),
                   jax.ShapeDtypeStruct((N, D), jnp.float32)],
    )(h, Wt, b2, dinvc)


def _tc_comb_lin_body(a_ref, z_ref, xp_ref, dinv_ref, wt_ref, b_ref,
                      xpo_ref, y_ref):
    s = a_ref[0] + a_ref[1] + z_ref[0] + z_ref[1]
    h = jnp.maximum(dinv_ref[...] * s + xp_ref[...], 0.0)
    xp = jnp.dot(h, wt_ref[...],
                 preferred_element_type=jnp.float32) + b_ref[...]
    xpo_ref[...] = xp
    y_ref[...] = xp * dinv_ref[...]


def _tc_comb_lin(apart, zpart, xp, dinvc, Wt, b2):
    return pl.pallas_call(
        _tc_comb_lin_body,
        grid=(N // BN,),
        in_specs=[pl.BlockSpec((2, BN, D), lambda i: (0, i, 0)),
                  pl.BlockSpec((2, BN, D), lambda i: (0, i, 0)),
                  pl.BlockSpec((BN, D), lambda i: (i, 0)),
                  pl.BlockSpec((BN, 1), lambda i: (i, 0)),
                  pl.BlockSpec((D, D), lambda i: (0, 0)),
                  pl.BlockSpec((1, D), lambda i: (0, 0))],
        out_specs=[pl.BlockSpec((BN, D), lambda i: (i, 0)),
                   pl.BlockSpec((BN, D), lambda i: (i, 0))],
        out_shape=[jax.ShapeDtypeStruct((N, D), jnp.float32),
                   jax.ShapeDtypeStruct((N, D), jnp.float32)],
    )(apart, zpart, xp, dinvc, Wt, b2)


def _tc_final_body(a_ref, z_ref, xp_ref, dinv_ref, o_ref):
    s = a_ref[0] + a_ref[1] + z_ref[0] + z_ref[1]
    o_ref[...] = dinv_ref[...] * s + xp_ref[...]


def _tc_final(apart, zpart, xp, dinvc):
    return pl.pallas_call(
        _tc_final_body,
        grid=(N // BN,),
        in_specs=[pl.BlockSpec((2, BN, D), lambda i: (0, i, 0)),
                  pl.BlockSpec((2, BN, D), lambda i: (0, i, 0)),
                  pl.BlockSpec((BN, D), lambda i: (i, 0)),
                  pl.BlockSpec((BN, 1), lambda i: (i, 0))],
        out_specs=pl.BlockSpec((BN, D), lambda i: (i, 0)),
        out_shape=jax.ShapeDtypeStruct((N, D), jnp.float32),
    )(apart, zpart, xp, dinvc)


def kernel(x, adj_t, edge_attr, emb_ea, W0, b0, W1, b1, W2, b2):
    row = adj_t[0]
    col = adj_t[1]
    row2 = row.reshape(NW, NCH, CH)
    col2 = col.reshape(NW, NCH, CH)
    zeros = jnp.zeros((RP, D), jnp.float32)

    degp = _sc_degree(col).reshape(NW, N)
    dinv2 = _tc_dinv(degp)
    dinv = dinv2.reshape(N)
    dinvc = dinv2.reshape(N, 1)

    drow = _sc_drow(row, dinv)
    z = _tc_z(edge_attr, emb_ea, drow)
    zpart = _sc_scatter_linear(z, row2, col2, zeros).reshape(NC, NP, D)

    xp, y = _tc_lin(x, W0.T, b0.reshape(1, D), dinvc)
    for Wl, bl in ((W1, b1), (W2, b2)):
        apart = _sc_scatter_gather(y, row2, col2, zeros).reshape(NC, NP, D)
        xp, y = _tc_comb_lin(apart, zpart, xp, dinvc, Wl.T,
                             bl.reshape(1, D))
    apart = _sc_scatter_gather(y, row2, col2, zeros).reshape(NC, NP, D)
    return _tc_final(apart, zpart, xp, dinvc)
